# baseline (speedup 1.0000x reference)
import jax
import jax.numpy as jnp
from jax import lax
from jax.experimental import pallas as pl
from jax.experimental.pallas import tpu as pltpu
from jax.experimental.pallas import tpu_sc as plsc

_NC = 2
_NS = 16
_NW = _NC * _NS
_L = 16
_IDX_CHUNK = 128


def _gmf_body(user_hbm, item_hbm, ut_hbm, it_hbm, out_hbm,
              uidx_v, iidx_v, urows_v, irows_v, sem_u, sem_i):
    b_per_w = urows_v.shape[0]
    n_chunks = b_per_w // _IDX_CHUNK
    d = urows_v.shape[1]
    wid = lax.axis_index("s") * _NC + lax.axis_index("c")
    base = wid * b_per_w

    for j in range(n_chunks):
        pltpu.sync_copy(user_hbm.at[pl.ds(base + j * _IDX_CHUNK, _IDX_CHUNK)],
                        uidx_v.at[j])
        pltpu.sync_copy(item_hbm.at[pl.ds(base + j * _IDX_CHUNK, _IDX_CHUNK)],
                        iidx_v.at[j])

    copies = []
    for j in range(n_chunks):
        dst = pl.ds(j * _IDX_CHUNK, _IDX_CHUNK)
        copies.append(pltpu.async_copy(ut_hbm.at[uidx_v.at[j]],
                                       urows_v.at[dst], sem_u))
        copies.append(pltpu.async_copy(it_hbm.at[iidx_v.at[j]],
                                       irows_v.at[dst], sem_i))
    for cp in copies:
        cp.wait()

    def mul_row(i, _):
        for h in range(d // _L):
            sl = pl.ds(h * _L, _L)
            urows_v[i, sl] = urows_v[i, sl] * irows_v[i, sl]
        return ()
    lax.fori_loop(0, b_per_w, mul_row, ())

    pltpu.sync_copy(urows_v, out_hbm.at[pl.ds(base, b_per_w)])


@jax.jit
def kernel(user, item, user_table, item_table):
    b = user.shape[0]
    d = user_table.shape[1]
    b_per_w = b // _NW
    n_chunks = b_per_w // _IDX_CHUNK
    mesh = plsc.VectorSubcoreMesh(core_axis_name="c", subcore_axis_name="s")
    k = pl.kernel(
        _gmf_body,
        out_type=jax.ShapeDtypeStruct((b, d), jnp.float32),
        mesh=mesh,
        compiler_params=pltpu.CompilerParams(use_tc_tiling_on_sc=False),
        scratch_types=[
            pltpu.VMEM((n_chunks, _IDX_CHUNK), jnp.int32),
            pltpu.VMEM((n_chunks, _IDX_CHUNK), jnp.int32),
            pltpu.VMEM((b_per_w, d), jnp.float32),
            pltpu.VMEM((b_per_w, d), jnp.float32),
            pltpu.SemaphoreType.DMA,
            pltpu.SemaphoreType.DMA,
        ],
    )
    return k(user.astype(jnp.int32), item.astype(jnp.int32),
             user_table, item_table)


# speedup vs baseline: 1.3945x; 1.3945x over previous
import jax
import jax.numpy as jnp
from jax import lax
from jax.experimental import pallas as pl
from jax.experimental.pallas import tpu as pltpu
from jax.experimental.pallas import tpu_sc as plsc

_NC = 2
_NS = 16
_NW = _NC * _NS
_L = 16
_CHUNK = 16
_NBUF = 2


def _gmf_body(user_hbm, item_hbm, ut_hbm, it_hbm, out_hbm,
              su_v, si_v, ut_tiles, it_tiles, out_stage,
              sem_u0, sem_u1, sem_i0, sem_i1, sem_o0, sem_o1):
    sems_u = (sem_u0, sem_u1)
    sems_i = (sem_i0, sem_i1)
    sems_o = (sem_o0, sem_o1)
    b_per_w = su_v.shape[0]
    n_chunks = b_per_w // _CHUNK
    wid = lax.axis_index("s") * _NC + lax.axis_index("c")
    base = wid * b_per_w

    ut3 = ut_hbm.reshape(ut_hbm.shape[0] // 8, 8, 32)
    it3 = it_hbm.reshape(it_hbm.shape[0] // 8, 8, 32)
    out3 = out_hbm.reshape(out_hbm.shape[0] // 8, 8, 32)

    pltpu.sync_copy(user_hbm.at[pl.ds(base, b_per_w)], su_v)
    pltpu.sync_copy(item_hbm.at[pl.ds(base, b_per_w)], si_v)

    def fire(c, buf):
        qu_vec = su_v[pl.ds(c * _CHUNK, _L)] >> 3
        qi_vec = si_v[pl.ds(c * _CHUNK, _L)] >> 3
        for j in range(_CHUNK):
            pltpu.async_copy(ut3.at[pl.ds(qu_vec[j], 1)],
                             ut_tiles.at[buf].at[pl.ds(j, 1)], sems_u[buf])
            pltpu.async_copy(it3.at[pl.ds(qi_vec[j], 1)],
                             it_tiles.at[buf].at[pl.ds(j, 1)], sems_i[buf])

    def drain(buf):
        for j in range(_CHUNK):
            pltpu.make_async_copy(ut3.at[pl.ds(0, 1)],
                                  ut_tiles.at[buf].at[pl.ds(j, 1)],
                                  sems_u[buf]).wait()
            pltpu.make_async_copy(it3.at[pl.ds(0, 1)],
                                  it_tiles.at[buf].at[pl.ds(j, 1)],
                                  sems_i[buf]).wait()

    lanes = lax.iota(jnp.int32, _L)

    def extract(c, buf):
        su = su_v[pl.ds(c * _CHUNK, _L)] & 7
        si = si_v[pl.ds(c * _CHUNK, _L)] & 7
        g = lanes >> 3
        s = lanes & 7
        def per_f(f, _):
            fv = jnp.full((_L,), f, jnp.int32)
            u = plsc.load_gather(ut_tiles.at[buf], [lanes, su, fv])
            v = plsc.load_gather(it_tiles.at[buf], [lanes, si, fv])
            plsc.store_scatter(out_stage.at[buf], [g, s, fv], u * v)
            return ()
        lax.fori_loop(0, 32, per_f, ())

    def flush(c, buf):
        pltpu.async_copy(out_stage.at[buf],
                         out3.at[pl.ds(base // 8 + c * (_CHUNK // 8),
                                       _CHUNK // 8)], sems_o[buf])

    def drain_out(buf):
        pltpu.make_async_copy(out_stage.at[buf],
                              out3.at[pl.ds(0, _CHUNK // 8)],
                              sems_o[buf]).wait()

    fire(0, 0)

    def step(c2, _):
        for p in range(_NBUF):
            c = c2 * _NBUF + p
            nxt = c + 1
            @pl.when(nxt < n_chunks)
            def _():
                fire(nxt, (p + 1) % _NBUF)
            drain(p)
            @pl.when(c >= _NBUF)
            def _():
                drain_out(p)
            extract(c, p)
            flush(c, p)
        return ()
    lax.fori_loop(0, n_chunks // _NBUF, step, ())
    for p in range(_NBUF):
        drain_out(p)


@jax.jit
def kernel(user, item, user_table, item_table):
    b = user.shape[0]
    d = user_table.shape[1]
    b_per_w = b // _NW
    mesh = plsc.VectorSubcoreMesh(core_axis_name="c", subcore_axis_name="s")
    k = pl.kernel(
        _gmf_body,
        out_type=jax.ShapeDtypeStruct((b, d), jnp.float32),
        mesh=mesh,
        compiler_params=pltpu.CompilerParams(use_tc_tiling_on_sc=True,
                                             needs_layout_passes=False),
        scratch_types=[
            pltpu.VMEM((b_per_w,), jnp.int32),
            pltpu.VMEM((b_per_w,), jnp.int32),
            pltpu.VMEM((_NBUF, _CHUNK, 8, 32), jnp.float32),
            pltpu.VMEM((_NBUF, _CHUNK, 8, 32), jnp.float32),
            pltpu.VMEM((_NBUF, _CHUNK // 8, 8, 32), jnp.float32),
            pltpu.SemaphoreType.DMA,
            pltpu.SemaphoreType.DMA,
            pltpu.SemaphoreType.DMA,
            pltpu.SemaphoreType.DMA,
            pltpu.SemaphoreType.DMA,
            pltpu.SemaphoreType.DMA,
        ],
    )
    return k(user.astype(jnp.int32), item.astype(jnp.int32),
             user_table, item_table)


# speedup vs baseline: 2.2739x; 1.6306x over previous
import jax
import jax.numpy as jnp
from jax import lax
from jax.experimental import pallas as pl
from jax.experimental.pallas import tpu as pltpu
from jax.experimental.pallas import tpu_sc as plsc

_NC = 2
_NS = 16
_NW = _NC * _NS
_L = 16
_CHUNK = 16
_NBUF = 2


def _gmf_body(user_hbm, item_hbm, ut_hbm, it_hbm, out_hbm,
              su_v, si_v, ut_tiles, it_tiles, out_stage,
              sem_u0, sem_u1, sem_i0, sem_i1, sem_o0, sem_o1):
    sems_u = (sem_u0, sem_u1)
    sems_i = (sem_i0, sem_i1)
    sems_o = (sem_o0, sem_o1)
    b_per_w = su_v.shape[0]
    n_chunks = b_per_w // _CHUNK
    wid = lax.axis_index("s") * _NC + lax.axis_index("c")
    base = wid * b_per_w

    ut3 = ut_hbm
    it3 = it_hbm
    out3 = out_hbm.reshape(out_hbm.shape[0] // 8, 8, 32)

    pltpu.sync_copy(user_hbm.at[pl.ds(base, b_per_w)], su_v)
    pltpu.sync_copy(item_hbm.at[pl.ds(base, b_per_w)], si_v)

    def fire(c, buf):
        qu_vec = su_v[pl.ds(c * _CHUNK, _L)] >> 3
        qi_vec = si_v[pl.ds(c * _CHUNK, _L)] >> 3
        for j in range(_CHUNK):
            pltpu.async_copy(ut3.at[pl.ds(qu_vec[j], 1)],
                             ut_tiles.at[buf].at[pl.ds(j, 1)], sems_u[buf])
            pltpu.async_copy(it3.at[pl.ds(qi_vec[j], 1)],
                             it_tiles.at[buf].at[pl.ds(j, 1)], sems_i[buf])

    def drain(buf):
        for j in range(_CHUNK):
            pltpu.make_async_copy(ut3.at[pl.ds(0, 1)],
                                  ut_tiles.at[buf].at[pl.ds(j, 1)],
                                  sems_u[buf]).wait()
            pltpu.make_async_copy(it3.at[pl.ds(0, 1)],
                                  it_tiles.at[buf].at[pl.ds(j, 1)],
                                  sems_i[buf]).wait()

    lanes = lax.iota(jnp.int32, _L)

    def extract(c, buf):
        su = su_v[pl.ds(c * _CHUNK, _L)] & 7
        si = si_v[pl.ds(c * _CHUNK, _L)] & 7
        g = lanes >> 3
        s = lanes & 7
        def per_f(f, _):
            fv = jnp.full((_L,), f, jnp.int32)
            u = plsc.load_gather(ut_tiles.at[buf], [lanes, su, fv])
            v = plsc.load_gather(it_tiles.at[buf], [lanes, si, fv])
            plsc.store_scatter(out_stage.at[buf], [g, s, fv], u * v)
            return ()
        lax.fori_loop(0, 32, per_f, ())

    def flush(c, buf):
        pltpu.async_copy(out_stage.at[buf],
                         out3.at[pl.ds(base // 8 + c * (_CHUNK // 8),
                                       _CHUNK // 8)], sems_o[buf])

    def drain_out(buf):
        pltpu.make_async_copy(out_stage.at[buf],
                              out3.at[pl.ds(0, _CHUNK // 8)],
                              sems_o[buf]).wait()

    fire(0, 0)

    def step(c2, _):
        for p in range(_NBUF):
            c = c2 * _NBUF + p
            nxt = c + 1
            @pl.when(nxt < n_chunks)
            def _():
                fire(nxt, (p + 1) % _NBUF)
            drain(p)
            @pl.when(c >= _NBUF)
            def _():
                drain_out(p)
            extract(c, p)
            flush(c, p)
        return ()
    lax.fori_loop(0, n_chunks // _NBUF, step, ())
    for p in range(_NBUF):
        drain_out(p)


@jax.jit
def kernel(user, item, user_table, item_table):
    b = user.shape[0]
    d = user_table.shape[1]
    b_per_w = b // _NW
    mesh = plsc.VectorSubcoreMesh(core_axis_name="c", subcore_axis_name="s")
    k = pl.kernel(
        _gmf_body,
        out_type=jax.ShapeDtypeStruct((b, d), jnp.float32),
        mesh=mesh,
        compiler_params=pltpu.CompilerParams(use_tc_tiling_on_sc=True,
                                             needs_layout_passes=False),
        scratch_types=[
            pltpu.VMEM((b_per_w,), jnp.int32),
            pltpu.VMEM((b_per_w,), jnp.int32),
            pltpu.VMEM((_NBUF, _CHUNK, 8, 32), jnp.float32),
            pltpu.VMEM((_NBUF, _CHUNK, 8, 32), jnp.float32),
            pltpu.VMEM((_NBUF, _CHUNK // 8, 8, 32), jnp.float32),
            pltpu.SemaphoreType.DMA,
            pltpu.SemaphoreType.DMA,
            pltpu.SemaphoreType.DMA,
            pltpu.SemaphoreType.DMA,
            pltpu.SemaphoreType.DMA,
            pltpu.SemaphoreType.DMA,
        ],
    )
    return k(user.astype(jnp.int32), item.astype(jnp.int32),
             jnp.reshape(user_table, (user_table.shape[0] // 8, 8, d)),
             jnp.reshape(item_table, (item_table.shape[0] // 8, 8, d)))
